# Initial kernel scaffold; baseline (speedup 1.0000x reference)
import jax
import jax.numpy as jnp
from jax.experimental import pallas as pl


def kernel(lS_i, W):
    raise NotImplementedError("write your pallas kernel here")


# speedup vs baseline: 7.0260x; 7.0260x over previous
import jax
import jax.numpy as jnp
from jax import lax
from jax.experimental import pallas as pl
from jax.experimental.pallas import tpu as pltpu
from jax.experimental.pallas import tpu_sc as plsc

NUM_TABLES = 26
VOCAB = 100000
DIM = 32
BATCH = 4096
HIST = 20

NC = 2
NS = 16
NW = NC * NS
BAGS_W = BATCH // NW
IDX_W = BAGS_W * HIST
GCH = 128
NG = IDX_W // GCH


def _emb_body(idx_hbm, w_hbm, out_hbm, idxv, rowsv, outv, sem):
    wid = lax.axis_index("s") * NC + lax.axis_index("c")

    def table_body(t, carry):
        base_bag = t * BATCH + wid * BAGS_W
        pltpu.sync_copy(idx_hbm.at[pl.ds(base_bag * HIST, IDX_W)], idxv)

        off = t * VOCAB

        def add_off(i, c):
            idxv[pl.ds(i * 16, 16)] = idxv[pl.ds(i * 16, 16)] + off
            return c

        lax.fori_loop(0, IDX_W // 16, add_off, 0)

        cps = [
            pltpu.async_copy(
                w_hbm.at[idxv.at[pl.ds(k * GCH, GCH)]],
                rowsv.at[pl.ds(k * GCH, GCH)],
                sem,
            )
            for k in range(NG)
        ]
        for cp in cps:
            cp.wait()

        def bag_body(b, c):
            r = b * HIST
            a0 = rowsv[r, pl.ds(0, 16)]
            a1 = rowsv[r, pl.ds(16, 16)]
            for j in range(1, HIST):
                a0 = a0 + rowsv[r + j, pl.ds(0, 16)]
                a1 = a1 + rowsv[r + j, pl.ds(16, 16)]
            outv[b, pl.ds(0, 16)] = a0
            outv[b, pl.ds(16, 16)] = a1
            return c

        lax.fori_loop(0, BAGS_W, bag_body, 0)

        pltpu.sync_copy(outv, out_hbm.at[pl.ds(base_bag, BAGS_W)])
        return carry

    lax.fori_loop(0, NUM_TABLES, table_body, 0)


def kernel(lS_i, W):
    idx_flat = lS_i.reshape(NUM_TABLES * BATCH * HIST).astype(jnp.int32)
    w_flat = W.reshape(NUM_TABLES * VOCAB, DIM)
    mesh = plsc.VectorSubcoreMesh(core_axis_name="c", subcore_axis_name="s")
    run = pl.kernel(
        _emb_body,
        mesh=mesh,
        compiler_params=pltpu.CompilerParams(use_tc_tiling_on_sc=False),
        out_type=jax.ShapeDtypeStruct((NUM_TABLES * BATCH, DIM), jnp.float32),
        scratch_types=[
            pltpu.VMEM((IDX_W,), jnp.int32),
            pltpu.VMEM((IDX_W, DIM), jnp.float32),
            pltpu.VMEM((BAGS_W, DIM), jnp.float32),
            pltpu.SemaphoreType.DMA,
        ],
    )
    out = run(idx_flat, w_flat)
    return out.reshape(NUM_TABLES, BATCH, DIM)


# speedup vs baseline: 27.1023x; 3.8574x over previous
import jax
import jax.numpy as jnp
from jax import lax
from jax.experimental import pallas as pl
from jax.experimental.pallas import tpu as pltpu
from jax.experimental.pallas import tpu_sc as plsc

NUM_TABLES = 26
VOCAB = 100000
DIM = 32
BATCH = 4096
HIST = 20

NC = 2
NS = 16
NW = NC * NS
PIECE = 512
NPIECE = BATCH // PIECE
NCHUNK = PIECE // 16


def _emb_body(idx_hbm, w_hbm, out_hbm, idxp, wcol, outv, sem0, sem1, semo):
    col = lax.axis_index("s") * NC + lax.axis_index("c")
    off = (col * NUM_TABLES) // NW
    sems = (sem0, sem1)

    def fetch_idx(t, p, buf):
        return pltpu.async_copy(
            idx_hbm.at[t, :, pl.ds(p * PIECE, PIECE)], idxp.at[buf], sems[buf]
        )

    fetch_idx(off, 0, 0)

    def table_body(tl, carry):
        t = lax.rem(tl + off, NUM_TABLES)
        tn = lax.rem(tl + 1 + off, NUM_TABLES)
        pltpu.sync_copy(w_hbm.at[t, col], wcol)

        @pl.when(tl > 0)
        def _():
            pltpu.make_async_copy(out_hbm.at[t, col], outv, semo).wait()

        for p in range(NPIECE):
            buf = p % 2
            nbuf = (p + 1) % 2
            if p < NPIECE - 1:
                fetch_idx(t, p + 1, nbuf)
            else:
                @pl.when(tl < NUM_TABLES - 1)
                def _():
                    fetch_idx(tn, 0, 0)

            pltpu.make_async_copy(
                idx_hbm.at[t, :, pl.ds(p * PIECE, PIECE)], idxp.at[buf], sems[buf]
            ).wait()

            def chunk_body(c, cc):
                for u in range(2):
                    ci = c * 2 + u
                    lanes = pl.ds(ci * 16, 16)
                    accs = [
                        plsc.load_gather(wcol, [idxp[buf, j, lanes]])
                        for j in range(4)
                    ]
                    for j in range(4, HIST):
                        accs[j % 4] = accs[j % 4] + plsc.load_gather(
                            wcol, [idxp[buf, j, lanes]]
                        )
                    outv[pl.ds(p * PIECE + ci * 16, 16)] = (
                        (accs[0] + accs[1]) + (accs[2] + accs[3])
                    )
                return cc

            lax.fori_loop(0, NCHUNK // 2, chunk_body, 0)

        pltpu.async_copy(outv, out_hbm.at[t, col], semo)
        return carry

    lax.fori_loop(0, NUM_TABLES, table_body, 0)
    pltpu.make_async_copy(
        out_hbm.at[lax.rem(NUM_TABLES - 1 + off, NUM_TABLES), col], outv, semo
    ).wait()


def kernel(lS_i, W):
    idx_t = jnp.swapaxes(lS_i.astype(jnp.int32), 1, 2)
    w_t = jnp.swapaxes(W, 1, 2)
    mesh = plsc.VectorSubcoreMesh(core_axis_name="c", subcore_axis_name="s")
    run = pl.kernel(
        _emb_body,
        mesh=mesh,
        compiler_params=pltpu.CompilerParams(needs_layout_passes=False),
        out_type=jax.ShapeDtypeStruct((NUM_TABLES, DIM, BATCH), jnp.float32),
        scratch_types=[
            pltpu.VMEM((2, HIST, PIECE), jnp.int32),
            pltpu.VMEM((VOCAB,), jnp.float32),
            pltpu.VMEM((BATCH,), jnp.float32),
            pltpu.SemaphoreType.DMA,
            pltpu.SemaphoreType.DMA,
            pltpu.SemaphoreType.DMA,
        ],
    )
    out_t = run(idx_t, w_t)
    return jnp.swapaxes(out_t, 1, 2)


# speedup vs baseline: 28.4837x; 1.0510x over previous
import jax
import jax.numpy as jnp
from jax import lax
from jax.experimental import pallas as pl
from jax.experimental.pallas import tpu as pltpu
from jax.experimental.pallas import tpu_sc as plsc

NUM_TABLES = 26
VOCAB = 100000
DIM = 32
BATCH = 4096
HIST = 20

NC = 2
NS = 16
NW = NC * NS
PIECE = 512
NPIECE = BATCH // PIECE
NCHUNK = PIECE // 16


def _emb_body(idx_hbm, w_hbm, out_hbm, idxp, wcol, outv, sem0, sem1, semo):
    col = lax.axis_index("s") * NC + lax.axis_index("c")
    off = (col * NUM_TABLES) // NW
    sems = (sem0, sem1)

    def fetch_idx(t, p, buf):
        return pltpu.async_copy(
            idx_hbm.at[t, :, pl.ds(p * PIECE, PIECE)], idxp.at[buf], sems[buf]
        )

    fetch_idx(off, 0, 0)

    def table_body(tl, carry):
        t = lax.rem(tl + off, NUM_TABLES)
        tn = lax.rem(tl + 1 + off, NUM_TABLES)
        pltpu.sync_copy(w_hbm.at[t, col], wcol)

        @pl.when(tl > 0)
        def _():
            pltpu.make_async_copy(out_hbm.at[t, col], outv, semo).wait()

        for p in range(NPIECE):
            buf = p % 2
            nbuf = (p + 1) % 2
            if p < NPIECE - 1:
                fetch_idx(t, p + 1, nbuf)
            else:
                @pl.when(tl < NUM_TABLES - 1)
                def _():
                    fetch_idx(tn, 0, 0)

            pltpu.make_async_copy(
                idx_hbm.at[t, :, pl.ds(p * PIECE, PIECE)], idxp.at[buf], sems[buf]
            ).wait()

            @plsc.parallel_loop(0, NCHUNK, unroll=3)
            def _(c):
                lanes = pl.ds(c * 16, 16)
                accs = [
                    plsc.load_gather(wcol, [idxp[buf, j, lanes]])
                    for j in range(4)
                ]
                for j in range(4, HIST):
                    accs[j % 4] = accs[j % 4] + plsc.load_gather(
                        wcol, [idxp[buf, j, lanes]]
                    )
                outv[pl.ds(p * PIECE + c * 16, 16)] = (
                    (accs[0] + accs[1]) + (accs[2] + accs[3])
                )

        pltpu.async_copy(outv, out_hbm.at[t, col], semo)
        return carry

    lax.fori_loop(0, NUM_TABLES, table_body, 0)
    pltpu.make_async_copy(
        out_hbm.at[lax.rem(NUM_TABLES - 1 + off, NUM_TABLES), col], outv, semo
    ).wait()


def kernel(lS_i, W):
    idx_t = jnp.swapaxes(lS_i.astype(jnp.int32), 1, 2)
    w_t = jnp.swapaxes(W, 1, 2)
    mesh = plsc.VectorSubcoreMesh(core_axis_name="c", subcore_axis_name="s")
    run = pl.kernel(
        _emb_body,
        mesh=mesh,
        compiler_params=pltpu.CompilerParams(needs_layout_passes=False),
        out_type=jax.ShapeDtypeStruct((NUM_TABLES, DIM, BATCH), jnp.float32),
        scratch_types=[
            pltpu.VMEM((2, HIST, PIECE), jnp.int32),
            pltpu.VMEM((VOCAB,), jnp.float32),
            pltpu.VMEM((BATCH,), jnp.float32),
            pltpu.SemaphoreType.DMA,
            pltpu.SemaphoreType.DMA,
            pltpu.SemaphoreType.DMA,
        ],
    )
    out_t = run(idx_t, w_t)
    return jnp.swapaxes(out_t, 1, 2)
